# baseline (speedup 1.0000x reference)
import functools

import jax
import jax.numpy as jnp
from jax import lax
from jax.experimental import pallas as pl
from jax.experimental.pallas import tpu as pltpu
from jax.experimental.pallas import tpu_sc as plsc

INPUT_DIM = 768
HIDDEN_DIM = 384
NUM_EXPERTS = 64
TOP_K = 8
CHUNKS = 4
_INV_SQRT2 = 0.7071067811865476


def _gate_body(x_ref, w1_ref, b1_ref, w2t_ref, b2_ref, acc_in_ref,
               logits_ref, loss_ref, acc_out_ref, acc_ref,
               *, n_tokens, is_last):
    i = pl.program_id(0)
    nsteps = pl.num_programs(0)

    x = x_ref[...]
    h = jnp.dot(x, w1_ref[...], preferred_element_type=jnp.float32)
    h = h + b1_ref[...]
    h = 0.5 * h * (1.0 + lax.erf(h * _INV_SQRT2))
    logits_t = lax.dot_general(
        w2t_ref[...], h,
        dimension_numbers=(((1,), (1,)), ((), ())),
        preferred_element_type=jnp.float32,
    )
    logits_t = logits_t + b2_ref[...]
    logits_ref[...] = logits_t

    t = logits_t.shape[1]
    m_all = jnp.max(logits_t, axis=0, keepdims=True)
    p = jnp.exp(logits_t - m_all)
    probs = p * (1.0 / jnp.sum(p, axis=0, keepdims=True))

    lanes = acc_ref.shape[1]
    psum = probs[:, 0:lanes]
    for c in range(1, t // lanes):
        psum = psum + probs[:, c * lanes:(c + 1) * lanes]

    @pl.when(i == 0)
    def _():
        acc_ref[...] = acc_in_ref[...]

    acc_ref[...] += psum

    @pl.when(i == nsteps - 1)
    def _():
        acc_out_ref[...] = acc_ref[...]
        if is_last:
            mean_probs = jnp.sum(acc_ref[...], axis=1, keepdims=True) * (
                1.0 / n_tokens)
            diff = mean_probs - (1.0 / NUM_EXPERTS)
            loss_ref[...] = jnp.sum(
                diff * diff, keepdims=True).reshape(1, 1) * (1.0 / NUM_EXPERTS)
        else:
            loss_ref[...] = jnp.zeros_like(loss_ref)


def _tc_chunk(x_chunk, w1, b1r, w2t, b2r, acc_in, n_tokens, is_last):
    nc = x_chunk.shape[0]
    block_t = min(4096, nc)
    grid = (nc // block_t,)
    return pl.pallas_call(
        functools.partial(_gate_body, n_tokens=n_tokens, is_last=is_last),
        grid=grid,
        in_specs=[
            pl.BlockSpec((block_t, INPUT_DIM), lambda i: (i, 0)),
            pl.BlockSpec((INPUT_DIM, HIDDEN_DIM), lambda i: (0, 0)),
            pl.BlockSpec((1, HIDDEN_DIM), lambda i: (0, 0)),
            pl.BlockSpec((NUM_EXPERTS, HIDDEN_DIM), lambda i: (0, 0)),
            pl.BlockSpec((NUM_EXPERTS, 1), lambda i: (0, 0)),
            pl.BlockSpec((NUM_EXPERTS, 128), lambda i: (0, 0)),
        ],
        out_specs=[
            pl.BlockSpec((NUM_EXPERTS, block_t), lambda i: (0, i)),
            pl.BlockSpec((1, 1), lambda i: (0, 0)),
            pl.BlockSpec((NUM_EXPERTS, 128), lambda i: (0, 0)),
        ],
        out_shape=[
            jax.ShapeDtypeStruct((NUM_EXPERTS, nc), jnp.float32),
            jax.ShapeDtypeStruct((1, 1), jnp.float32),
            jax.ShapeDtypeStruct((NUM_EXPERTS, 128), jnp.float32),
        ],
        scratch_shapes=[pltpu.VMEM((NUM_EXPERTS, 128), jnp.float32)],
    )(x_chunk, w1, b1r, w2t, b2r, acc_in)


def _make_sc_topk(n):
    info = plsc.get_sparse_core_info()
    nc, ns, lanes = info.num_cores, info.num_subcores, info.num_lanes
    nw = nc * ns
    per = n // nw
    groups = per // lanes
    mesh = plsc.VectorSubcoreMesh(core_axis_name="c", subcore_axis_name="s")

    @functools.partial(
        pl.kernel, mesh=mesh,
        out_type=[
            jax.ShapeDtypeStruct((TOP_K, n), jnp.float32),
            jax.ShapeDtypeStruct((TOP_K, n), jnp.int32),
        ],
        scratch_types=[
            pltpu.VMEM((NUM_EXPERTS, per), jnp.float32),
            pltpu.VMEM((TOP_K, per), jnp.float32),
            pltpu.VMEM((TOP_K, per), jnp.int32),
        ],
    )
    def sc_topk(logits_hbm, gates_hbm, idx_hbm, lg_v, gv, iv):
        wid = lax.axis_index("s") * nc + lax.axis_index("c")
        base = wid * per
        pltpu.sync_copy(logits_hbm.at[:, pl.ds(base, per)], lg_v)

        def group_body(g, carry):
            t0 = g * lanes
            neg = jnp.full((lanes,), -jnp.inf, jnp.float32)
            zero_i = jnp.zeros((lanes,), jnp.int32)
            vals = [neg] * TOP_K
            idxs = [zero_i] * TOP_K
            for e in range(NUM_EXPERTS):
                nv = lg_v[e, pl.ds(t0, lanes)]
                ni = jnp.full((lanes,), e, jnp.int32)
                for j in range(TOP_K):
                    c = nv > vals[j]
                    new_v = jnp.where(c, nv, vals[j])
                    nv = jnp.where(c, vals[j], nv)
                    vals[j] = new_v
                    new_i = jnp.where(c, ni, idxs[j])
                    ni = jnp.where(c, idxs[j], ni)
                    idxs[j] = new_i
            gs = [jnp.exp(v - vals[0]) for v in vals]
            den = gs[0]
            for j in range(1, TOP_K):
                den = den + gs[j]
            rden = 1.0 / den
            for j in range(TOP_K):
                gv[j, pl.ds(t0, lanes)] = gs[j] * rden
                iv[j, pl.ds(t0, lanes)] = idxs[j]
            return carry

        lax.fori_loop(0, groups, group_body, 0)
        pltpu.sync_copy(gv, gates_hbm.at[:, pl.ds(base, per)])
        pltpu.sync_copy(iv, idx_hbm.at[:, pl.ds(base, per)])

    return sc_topk


def kernel(x, W1, b1, W2, b2, training=0):
    n = x.shape[0] * x.shape[1]
    x_flat = x.reshape(n, x.shape[2])
    nc = n // CHUNKS
    b1r = b1.reshape(1, HIDDEN_DIM)
    b2r = b2.reshape(NUM_EXPERTS, 1)
    w2t = W2.T

    sc_topk = _make_sc_topk(nc)
    acc = jnp.zeros((NUM_EXPERTS, 128), jnp.float32)
    gates_parts, idx_parts, loss = [], [], None
    for k in range(CHUNKS):
        xc = lax.slice_in_dim(x_flat, k * nc, (k + 1) * nc, axis=0)
        logits_t, loss_k, acc = _tc_chunk(
            xc, W1, b1r, w2t, b2r, acc, n, is_last=(k == CHUNKS - 1))
        g_t, i_t = sc_topk(logits_t)
        gates_parts.append(g_t)
        idx_parts.append(i_t)
        if k == CHUNKS - 1:
            loss = loss_k
    gates = jnp.concatenate(gates_parts, axis=1).T
    idx = jnp.concatenate(idx_parts, axis=1).T
    return gates, idx, loss[0, 0]


# speedup vs baseline: 2.1203x; 2.1203x over previous
import functools

import jax
import jax.numpy as jnp
from jax import lax
from jax.experimental import pallas as pl
from jax.experimental.pallas import tpu as pltpu

INPUT_DIM = 768
HIDDEN_DIM = 384
NUM_EXPERTS = 64
TOP_K = 8
_INV_SQRT2 = 0.7071067811865476


def _gate_body(x_ref, w1_ref, b1_ref, w2t_ref, b2_ref,
               gates_ref, idx_ref, loss_ref, acc_ref, *, n_tokens):
    i = pl.program_id(0)
    nsteps = pl.num_programs(0)

    x = x_ref[...]
    h = jnp.dot(x, w1_ref[...], preferred_element_type=jnp.float32)
    h = h + b1_ref[...]
    h = 0.5 * h * (1.0 + lax.erf(h * _INV_SQRT2))
    logits_t = lax.dot_general(
        w2t_ref[...], h,
        dimension_numbers=(((1,), (1,)), ((), ())),
        preferred_element_type=jnp.float32,
    )
    logits_t = logits_t + b2_ref[...]

    e = logits_t.shape[0]
    t = logits_t.shape[1]
    row_f = lax.broadcasted_iota(jnp.int32, (e, t), 0).astype(jnp.float32)

    def argmax_tree(v, im):
        rows = v.shape[0]
        while rows > 1:
            h = rows // 2
            c = v[0:h] >= v[h:rows]
            v = jnp.where(c, v[0:h], v[h:rows])
            im = jnp.where(c, im[0:h], im[h:rows])
            rows = h
        return v, im

    work = logits_t
    vals = []
    idxs = []
    for _ in range(TOP_K):
        mj, ij = argmax_tree(work, row_f)
        vals.append(mj)
        idxs.append(ij)
        work = jnp.where(row_f == ij, -jnp.inf, work)

    topv = jnp.concatenate(vals, axis=0)
    topi = jnp.concatenate(idxs, axis=0).astype(jnp.int32)

    g = jnp.exp(topv - topv[0:1, :])
    g = g * (1.0 / jnp.sum(g, axis=0, keepdims=True))
    gates_ref[...] = g.T
    idx_ref[...] = topi.T

    p = jnp.exp(logits_t - vals[0])
    probs = p * (1.0 / jnp.sum(p, axis=0, keepdims=True))

    lanes = acc_ref.shape[1]
    psum = probs[:, 0:lanes]
    for c in range(1, t // lanes):
        psum = psum + probs[:, c * lanes:(c + 1) * lanes]

    @pl.when(i == 0)
    def _():
        acc_ref[...] = jnp.zeros_like(acc_ref)

    acc_ref[...] += psum

    @pl.when(i == nsteps - 1)
    def _():
        mean_probs = jnp.sum(acc_ref[...], axis=1, keepdims=True) * (
            1.0 / n_tokens)
        diff = mean_probs - (1.0 / NUM_EXPERTS)
        loss_ref[...] = jnp.sum(diff * diff, keepdims=True).reshape(1, 1) * (
            1.0 / NUM_EXPERTS)


def kernel(x, W1, b1, W2, b2, training=0):
    n = x.shape[0] * x.shape[1]
    x_flat = x.reshape(n, x.shape[2])
    block_t = 4096
    grid = (n // block_t,)

    gates, idx, loss = pl.pallas_call(
        functools.partial(_gate_body, n_tokens=n),
        grid=grid,
        in_specs=[
            pl.BlockSpec((block_t, INPUT_DIM), lambda i: (i, 0)),
            pl.BlockSpec((INPUT_DIM, HIDDEN_DIM), lambda i: (0, 0)),
            pl.BlockSpec((1, HIDDEN_DIM), lambda i: (0, 0)),
            pl.BlockSpec((NUM_EXPERTS, HIDDEN_DIM), lambda i: (0, 0)),
            pl.BlockSpec((NUM_EXPERTS, 1), lambda i: (0, 0)),
        ],
        out_specs=[
            pl.BlockSpec((block_t, TOP_K), lambda i: (i, 0)),
            pl.BlockSpec((block_t, TOP_K), lambda i: (i, 0)),
            pl.BlockSpec((1, 1), lambda i: (0, 0)),
        ],
        out_shape=[
            jax.ShapeDtypeStruct((n, TOP_K), jnp.float32),
            jax.ShapeDtypeStruct((n, TOP_K), jnp.int32),
            jax.ShapeDtypeStruct((1, 1), jnp.float32),
        ],
        scratch_shapes=[pltpu.VMEM((NUM_EXPERTS, 128), jnp.float32)],
    )(x_flat, W1, b1.reshape(1, HIDDEN_DIM), W2.T,
      b2.reshape(NUM_EXPERTS, 1))

    return gates, idx, loss[0, 0]
